# Initial kernel scaffold; baseline (speedup 1.0000x reference)
import jax
import jax.numpy as jnp
from jax.experimental import pallas as pl


def kernel(x, edge_index, edge_attr):
    raise NotImplementedError("write your pallas kernel here")


# speedup vs baseline: 3.7288x; 3.7288x over previous
import functools

import jax
import jax.numpy as jnp
from jax import lax
from jax.experimental import pallas as pl
from jax.experimental.pallas import tpu as pltpu
from jax.experimental.pallas import tpu_sc as plsc

BETA = 0.2
GAMMA = 0.95
EPS = 1e-6

N = 10000
E = 320000
D = 128

NC = 2
NS = 16
HALF = D // NC
BLK = 128
NBLK = E // BLK
ACC_ROWS = 10240
ZCH = ACC_ROWS // NS
CCH = 80
NCCH = N // CCH


def _rowmax_tc(x):
    def body(x_ref, o_ref):
        o_ref[...] = GAMMA * jnp.max(x_ref[...], axis=1)[None, :]

    return pl.pallas_call(
        body,
        out_shape=jax.ShapeDtypeStruct((1, N), jnp.float32),
    )(x)


@functools.partial(
    pl.kernel,
    out_type=jax.ShapeDtypeStruct((NC, N, HALF), jnp.float32),
    mesh=plsc.VectorSubcoreMesh(core_axis_name="c", subcore_axis_name="s"),
    compiler_params=pltpu.CompilerParams(needs_layout_passes=False,
                                         use_tc_tiling_on_sc=False),
    scratch_types=[
        pltpu.VMEM((N,), jnp.float32),
        pltpu.VMEM((BLK,), jnp.int32),
        pltpu.VMEM((BLK,), jnp.int32),
        pltpu.VMEM((BLK,), jnp.float32),
        pltpu.VMEM((BLK, HALF), jnp.float32),
        pltpu.VMEM((BLK, HALF), jnp.float32),
        pltpu.VMEM((BLK, HALF), jnp.float32),
        pltpu.VMEM_SHARED((ACC_ROWS, HALF), jnp.float32),
        pltpu.VMEM_SHARED((ACC_ROWS, HALF), jnp.float32),
    ],
)
def _sc_gat(m_hbm, ei_hbm, ea_hbm, x_hbm, out_hbm,
            m_v, src_v, dst_v, s_v, ea0_v, ea1_v, x_v, acc_q, acc_ac):
    c = lax.axis_index("c")
    s = lax.axis_index("s")
    h0 = c * HALF

    def zrow(e, carry):
        z = jnp.zeros((16,), jnp.float32)
        for j in range(HALF // 16):
            ea0_v[e, pl.ds(j * 16, 16)] = z
        return carry

    lax.fori_loop(0, BLK, zrow, 0, unroll=4)
    for k in range(ZCH // BLK):
        r0 = s * ZCH + k * BLK
        pltpu.sync_copy(ea0_v, acc_q.at[pl.ds(r0, BLK)])
        pltpu.sync_copy(ea0_v, acc_ac.at[pl.ds(r0, BLK)])

    pltpu.sync_copy(m_hbm.at[0], m_v)
    plsc.subcore_barrier()

    blo = (s * NBLK) // NS
    bhi = ((s + 1) * NBLK) // NS

    def eblk(b, carry):
        e0 = b * BLK
        pltpu.sync_copy(ei_hbm.at[0, pl.ds(e0, BLK)], src_v)
        pltpu.sync_copy(ei_hbm.at[1, pl.ds(e0, BLK)], dst_v)
        pltpu.sync_copy(ea_hbm.at[pl.ds(e0, BLK), 0, pl.ds(h0, HALF)], ea0_v)
        pltpu.sync_copy(ea_hbm.at[pl.ds(e0, BLK), 1, pl.ds(h0, HALF)], ea1_v)

        for i in range(BLK // 16):
            idx = src_v[pl.ds(i * 16, 16)]
            s_v[pl.ds(i * 16, 16)] = plsc.load_gather(m_v, [idx])

        def fma16(g, carry2):
            sv16 = s_v[pl.ds(g * 16, 16)]
            for t in range(16):
                e = g * 16 + t
                sv = sv16[t]
                for j in range(HALF // 16):
                    sl = pl.ds(j * 16, 16)
                    ea0_v[e, sl] = ea0_v[e, sl] + sv * ea1_v[e, sl]
            return carry2

        lax.fori_loop(0, BLK // 16, fma16, 0)

        pltpu.sync_copy(ea0_v, acc_q.at[dst_v], add=True)
        pltpu.sync_copy(ea1_v, acc_ac.at[dst_v], add=True)
        return carry

    lax.fori_loop(blo, bhi, eblk, 0)
    plsc.subcore_barrier()

    clo = (s * NCCH) // NS
    chi = ((s + 1) * NCCH) // NS

    def cblk(ch, carry):
        r0 = ch * CCH
        pltpu.sync_copy(acc_q.at[pl.ds(r0, CCH)], ea0_v.at[pl.ds(0, CCH)])
        pltpu.sync_copy(acc_ac.at[pl.ds(r0, CCH)], ea1_v.at[pl.ds(0, CCH)])
        pltpu.sync_copy(x_hbm.at[c, pl.ds(r0, CCH)], x_v.at[pl.ds(0, CCH)])

        def crow(e, carry2):
            for j in range(HALF // 16):
                sl = pl.ds(j * 16, 16)
                q = ea0_v[e, sl]
                ac = ea1_v[e, sl]
                xv = x_v[e, sl]
                x_v[e, sl] = BETA * xv + (1.0 - BETA) * q / (ac + EPS)
            return carry2

        lax.fori_loop(0, CCH, crow, 0, unroll=2)
        pltpu.sync_copy(x_v.at[pl.ds(0, CCH)],
                        out_hbm.at[c, pl.ds(r0, CCH)])
        return carry

    lax.fori_loop(clo, chi, cblk, 0)


def kernel(x, edge_index, edge_attr):
    m = _rowmax_tc(x)
    xs = x.reshape(N, NC, HALF).transpose(1, 0, 2)
    out = _sc_gat(m, edge_index, edge_attr, xs)
    return out.transpose(1, 0, 2).reshape(N, D)


# speedup vs baseline: 7.8657x; 2.1094x over previous
import functools

import jax
import jax.numpy as jnp
from jax import lax
from jax.experimental import pallas as pl
from jax.experimental.pallas import tpu as pltpu
from jax.experimental.pallas import tpu_sc as plsc

BETA = 0.2
GAMMA = 0.95
EPS = 1e-6

N = 10000
E = 320000
D = 128

NC = 2
NS = 16
HALF = D // NC
BLK = 128
NBLK = E // BLK
UNIF = NBLK // NS
TAILB = NBLK - UNIF * NS
NBUF = 2
ACC_ROWS = 10240
ZCH = ACC_ROWS // NS
CCH = 80
NCCH = N // CCH


def _rowmax_tc(x):
    def body(x_ref, o_ref):
        o_ref[...] = GAMMA * jnp.max(x_ref[...], axis=1)[None, :]

    return pl.pallas_call(
        body,
        out_shape=jax.ShapeDtypeStruct((1, N), jnp.float32),
    )(x)


@functools.partial(
    pl.kernel,
    out_type=jax.ShapeDtypeStruct((NC, N, HALF), jnp.float32),
    mesh=plsc.VectorSubcoreMesh(core_axis_name="c", subcore_axis_name="s"),
    compiler_params=pltpu.CompilerParams(needs_layout_passes=False,
                                         use_tc_tiling_on_sc=False),
    scratch_types=[
        pltpu.VMEM((N,), jnp.float32),
        tuple(pltpu.VMEM((2, BLK), jnp.int32) for _ in range(NBUF)),
        pltpu.VMEM((BLK,), jnp.float32),
        tuple(pltpu.VMEM((BLK, HALF), jnp.float32) for _ in range(NBUF)),
        tuple(pltpu.VMEM((BLK, HALF), jnp.float32) for _ in range(NBUF)),
        pltpu.VMEM_SHARED((ACC_ROWS, HALF), jnp.float32),
        pltpu.VMEM_SHARED((ACC_ROWS, HALF), jnp.float32),
        tuple(pltpu.SemaphoreType.DMA for _ in range(NBUF)),
        tuple(pltpu.SemaphoreType.DMA for _ in range(NBUF)),
    ],
)
def _sc_gat(m_hbm, ei_hbm, ea_hbm, x_hbm, out_hbm,
            m_v, idx_v, s_v, ea0_v, ea1_v, acc_q, acc_ac,
            sem_in, sem_sc):
    c = lax.axis_index("c")
    s = lax.axis_index("s")
    h0 = c * HALF

    def zrow(e, carry):
        z = jnp.zeros((16,), jnp.float32)
        for j in range(HALF // 16):
            ea0_v[0][e, pl.ds(j * 16, 16)] = z
        return carry

    lax.fori_loop(0, BLK, zrow, 0, unroll=4)
    for k in range(ZCH // BLK):
        r0 = s * ZCH + k * BLK
        pltpu.sync_copy(ea0_v[0], acc_q.at[pl.ds(r0, BLK)])
        pltpu.sync_copy(ea0_v[0], acc_ac.at[pl.ds(r0, BLK)])

    pltpu.sync_copy(m_hbm.at[0], m_v)
    plsc.subcore_barrier()

    def fire_in(b, q):
        e0 = b * BLK
        pltpu.async_copy(ei_hbm.at[:, pl.ds(e0, BLK)], idx_v[q], sem_in[q])
        pltpu.async_copy(ea_hbm.at[pl.ds(e0, BLK), 0, pl.ds(h0, HALF)],
                         ea0_v[q], sem_in[q])
        pltpu.async_copy(ea_hbm.at[pl.ds(e0, BLK), 1, pl.ds(h0, HALF)],
                         ea1_v[q], sem_in[q])

    def wait_in(q):
        pltpu.make_async_copy(ei_hbm.at[:, pl.ds(0, BLK)],
                              idx_v[q], sem_in[q]).wait()
        pltpu.make_async_copy(ea_hbm.at[pl.ds(0, BLK), 0, pl.ds(0, HALF)],
                              ea0_v[q], sem_in[q]).wait()
        pltpu.make_async_copy(ea_hbm.at[pl.ds(0, BLK), 1, pl.ds(0, HALF)],
                              ea1_v[q], sem_in[q]).wait()

    def compute(q):
        for i in range(BLK // 16):
            idx = idx_v[q][0, pl.ds(i * 16, 16)]
            s_v[pl.ds(i * 16, 16)] = plsc.load_gather(m_v, [idx])

        def fma16(g, carry):
            sv16 = s_v[pl.ds(g * 16, 16)]
            for t in range(16):
                e = g * 16 + t
                sv = sv16[t]
                for j in range(HALF // 16):
                    sl = pl.ds(j * 16, 16)
                    ea0_v[q][e, sl] = ea0_v[q][e, sl] + sv * ea1_v[q][e, sl]
            return carry

        lax.fori_loop(0, BLK // 16, fma16, 0)

    def fire_sc(q):
        pltpu.async_copy(ea0_v[q], acc_q.at[idx_v[q].at[1]], sem_sc[q],
                         add=True)
        pltpu.async_copy(ea1_v[q], acc_ac.at[idx_v[q].at[1]], sem_sc[q],
                         add=True)

    def wait_sc(q):
        pltpu.make_async_copy(ea_hbm.at[pl.ds(0, BLK), 0, pl.ds(0, HALF)],
                              ea0_v[q], sem_sc[q]).wait()
        pltpu.make_async_copy(ea_hbm.at[pl.ds(0, BLK), 1, pl.ds(0, HALF)],
                              ea1_v[q], sem_sc[q]).wait()

    @pl.when(s < TAILB)
    def _tail():
        e0 = s * BLK
        pltpu.sync_copy(ei_hbm.at[:, pl.ds(e0, BLK)], idx_v[0])
        pltpu.sync_copy(ea_hbm.at[pl.ds(e0, BLK), 0, pl.ds(h0, HALF)],
                        ea0_v[0])
        pltpu.sync_copy(ea_hbm.at[pl.ds(e0, BLK), 1, pl.ds(h0, HALF)],
                        ea1_v[0])
        compute(0)
        pltpu.sync_copy(ea0_v[0], acc_q.at[idx_v[0].at[1]], add=True)
        pltpu.sync_copy(ea1_v[0], acc_ac.at[idx_v[0].at[1]], add=True)

    b0 = TAILB + s * UNIF

    fire_in(b0, 0)
    fire_in(b0 + 1, 1)

    def step(b, q, prefetch):
        wait_in(q)
        compute(q)
        fire_sc(q)
        wait_sc(q)
        if prefetch:
            fire_in(b + NBUF, q)

    def miter(i, carry):
        b = b0 + NBUF * i
        for q in range(NBUF):
            step(b + q, q, True)
        return carry

    lax.fori_loop(0, UNIF // NBUF - 1, miter, 0)

    bl = b0 + UNIF - NBUF
    for q in range(NBUF):
        step(bl + q, q, False)

    plsc.subcore_barrier()

    clo = (s * NCCH) // NS
    chi = ((s + 1) * NCCH) // NS

    def cblk(ch, carry):
        r0 = ch * CCH
        pltpu.sync_copy(acc_q.at[pl.ds(r0, CCH)], ea0_v[0].at[pl.ds(0, CCH)])
        pltpu.sync_copy(acc_ac.at[pl.ds(r0, CCH)], ea1_v[0].at[pl.ds(0, CCH)])
        pltpu.sync_copy(x_hbm.at[c, pl.ds(r0, CCH)], ea0_v[1].at[pl.ds(0, CCH)])

        def crow(e, carry2):
            for j in range(HALF // 16):
                sl = pl.ds(j * 16, 16)
                q = ea0_v[0][e, sl]
                ac = ea1_v[0][e, sl]
                xv = ea0_v[1][e, sl]
                ea0_v[1][e, sl] = BETA * xv + (1.0 - BETA) * q / (ac + EPS)
            return carry2

        lax.fori_loop(0, CCH, crow, 0, unroll=2)
        pltpu.sync_copy(ea0_v[1].at[pl.ds(0, CCH)],
                        out_hbm.at[c, pl.ds(r0, CCH)])
        return carry

    lax.fori_loop(clo, chi, cblk, 0)


def kernel(x, edge_index, edge_attr):
    m = _rowmax_tc(x)
    xs = x.reshape(N, NC, HALF).transpose(1, 0, 2)
    out = _sc_gat(m, edge_index, edge_attr, xs)
    return out.transpose(1, 0, 2).reshape(N, D)


# speedup vs baseline: 8.7651x; 1.1143x over previous
import functools

import jax
import jax.numpy as jnp
from jax import lax
from jax.experimental import pallas as pl
from jax.experimental.pallas import tpu as pltpu
from jax.experimental.pallas import tpu_sc as plsc

BETA = 0.2
GAMMA = 0.95
EPS = 1e-6

N = 10000
E = 320000
D = 128

NC = 2
NS = 16
HALF = D // NC
BLK = 128
NBLK = E // BLK
UNIF = NBLK // NS
TAILB = NBLK - UNIF * NS
NBUF = 2
ACC_ROWS = 10240
ZCH = ACC_ROWS // NS
CCH = 80
NCCH = N // CCH


def _rowmax_tc(x):
    def body(x_ref, o_ref):
        o_ref[...] = GAMMA * jnp.max(x_ref[...], axis=1)[None, :]

    return pl.pallas_call(
        body,
        out_shape=jax.ShapeDtypeStruct((1, N), jnp.float32),
    )(x)


@functools.partial(
    pl.kernel,
    out_type=jax.ShapeDtypeStruct((N, D), jnp.float32),
    mesh=plsc.VectorSubcoreMesh(core_axis_name="c", subcore_axis_name="s"),
    compiler_params=pltpu.CompilerParams(needs_layout_passes=False,
                                         use_tc_tiling_on_sc=False),
    scratch_types=[
        pltpu.VMEM((N,), jnp.float32),
        tuple(pltpu.VMEM((2, BLK), jnp.int32) for _ in range(NBUF)),
        pltpu.VMEM((BLK,), jnp.float32),
        tuple(pltpu.VMEM((BLK, HALF), jnp.float32) for _ in range(NBUF)),
        tuple(pltpu.VMEM((BLK, HALF), jnp.float32) for _ in range(NBUF)),
        pltpu.VMEM_SHARED((ACC_ROWS, HALF), jnp.float32),
        pltpu.VMEM_SHARED((ACC_ROWS, HALF), jnp.float32),
        tuple(pltpu.SemaphoreType.DMA for _ in range(NBUF)),
        tuple(pltpu.SemaphoreType.DMA for _ in range(NBUF)),
    ],
)
def _sc_gat(m_hbm, ei_hbm, ea_hbm, x_hbm, out_hbm,
            m_v, idx_v, s_v, ea0_v, ea1_v, acc_q, acc_ac,
            sem_in, sem_sc):
    c = lax.axis_index("c")
    s = lax.axis_index("s")
    h0 = c * HALF

    def zrow(e, carry):
        z = jnp.zeros((16,), jnp.float32)
        for j in range(HALF // 16):
            ea0_v[0][e, pl.ds(j * 16, 16)] = z
        return carry

    lax.fori_loop(0, BLK, zrow, 0, unroll=4)
    for k in range(ZCH // BLK):
        r0 = s * ZCH + k * BLK
        pltpu.sync_copy(ea0_v[0], acc_q.at[pl.ds(r0, BLK)])
        pltpu.sync_copy(ea0_v[0], acc_ac.at[pl.ds(r0, BLK)])

    pltpu.sync_copy(m_hbm.at[0], m_v)
    plsc.subcore_barrier()

    def fire_in(b, q):
        e0 = b * BLK
        pltpu.async_copy(ei_hbm.at[:, pl.ds(e0, BLK)], idx_v[q], sem_in[q])
        pltpu.async_copy(ea_hbm.at[pl.ds(e0, BLK), 0, pl.ds(h0, HALF)],
                         ea0_v[q], sem_in[q])
        pltpu.async_copy(ea_hbm.at[pl.ds(e0, BLK), 1, pl.ds(h0, HALF)],
                         ea1_v[q], sem_in[q])

    def wait_in(q):
        pltpu.make_async_copy(ei_hbm.at[:, pl.ds(0, BLK)],
                              idx_v[q], sem_in[q]).wait()
        pltpu.make_async_copy(ea_hbm.at[pl.ds(0, BLK), 0, pl.ds(0, HALF)],
                              ea0_v[q], sem_in[q]).wait()
        pltpu.make_async_copy(ea_hbm.at[pl.ds(0, BLK), 1, pl.ds(0, HALF)],
                              ea1_v[q], sem_in[q]).wait()

    def compute(q):
        for i in range(BLK // 16):
            idx = idx_v[q][0, pl.ds(i * 16, 16)]
            s_v[pl.ds(i * 16, 16)] = plsc.load_gather(m_v, [idx])

        def fma16(g, carry):
            sv16 = s_v[pl.ds(g * 16, 16)]
            for t in range(16):
                e = g * 16 + t
                sv = sv16[t]
                for j in range(HALF // 16):
                    sl = pl.ds(j * 16, 16)
                    ea0_v[q][e, sl] = ea0_v[q][e, sl] + sv * ea1_v[q][e, sl]
            return carry

        lax.fori_loop(0, BLK // 16, fma16, 0)

    def fire_sc(q):
        pltpu.async_copy(ea0_v[q], acc_q.at[idx_v[q].at[1]], sem_sc[q],
                         add=True)
        pltpu.async_copy(ea1_v[q], acc_ac.at[idx_v[q].at[1]], sem_sc[q],
                         add=True)

    def wait_sc(q):
        pltpu.make_async_copy(ea_hbm.at[pl.ds(0, BLK), 0, pl.ds(0, HALF)],
                              ea0_v[q], sem_sc[q]).wait()
        pltpu.make_async_copy(ea_hbm.at[pl.ds(0, BLK), 1, pl.ds(0, HALF)],
                              ea1_v[q], sem_sc[q]).wait()

    @pl.when(s < TAILB)
    def _tail():
        e0 = s * BLK
        pltpu.sync_copy(ei_hbm.at[:, pl.ds(e0, BLK)], idx_v[0])
        pltpu.sync_copy(ea_hbm.at[pl.ds(e0, BLK), 0, pl.ds(h0, HALF)],
                        ea0_v[0])
        pltpu.sync_copy(ea_hbm.at[pl.ds(e0, BLK), 1, pl.ds(h0, HALF)],
                        ea1_v[0])
        compute(0)
        pltpu.sync_copy(ea0_v[0], acc_q.at[idx_v[0].at[1]], add=True)
        pltpu.sync_copy(ea1_v[0], acc_ac.at[idx_v[0].at[1]], add=True)

    b0 = TAILB + s * UNIF

    fire_in(b0, 0)
    fire_in(b0 + 1, 1)

    def step(b, q, prefetch):
        wait_in(q)
        compute(q)
        fire_sc(q)
        wait_sc(q)
        if prefetch:
            fire_in(b + NBUF, q)

    def miter(i, carry):
        b = b0 + NBUF * i
        for q in range(NBUF):
            step(b + q, q, True)
        return carry

    lax.fori_loop(0, UNIF // NBUF - 1, miter, 0)

    bl = b0 + UNIF - NBUF
    for q in range(NBUF):
        step(bl + q, q, False)

    plsc.subcore_barrier()

    clo = (s * NCCH) // NS
    chi = ((s + 1) * NCCH) // NS

    def cblk(ch, carry):
        r0 = ch * CCH
        pltpu.sync_copy(acc_q.at[pl.ds(r0, CCH)], ea0_v[0].at[pl.ds(0, CCH)])
        pltpu.sync_copy(acc_ac.at[pl.ds(r0, CCH)], ea1_v[0].at[pl.ds(0, CCH)])
        pltpu.sync_copy(x_hbm.at[pl.ds(r0, CCH), pl.ds(h0, HALF)],
                        ea0_v[1].at[pl.ds(0, CCH)])

        def crow(e, carry2):
            for j in range(HALF // 16):
                sl = pl.ds(j * 16, 16)
                q = ea0_v[0][e, sl]
                ac = ea1_v[0][e, sl]
                xv = ea0_v[1][e, sl]
                ea0_v[1][e, sl] = BETA * xv + (1.0 - BETA) * q / (ac + EPS)
            return carry2

        lax.fori_loop(0, CCH, crow, 0, unroll=2)
        pltpu.sync_copy(ea0_v[1].at[pl.ds(0, CCH)],
                        out_hbm.at[pl.ds(r0, CCH), pl.ds(h0, HALF)])
        return carry

    lax.fori_loop(clo, chi, cblk, 0)


def kernel(x, edge_index, edge_attr):
    m = _rowmax_tc(x)
    return _sc_gat(m, edge_index, edge_attr, x)
